# baseline (speedup 1.0000x reference)
import functools

import jax
import jax.numpy as jnp
from jax import lax
from jax.experimental import pallas as pl
from jax.experimental.pallas import tpu as pltpu
from jax.experimental.pallas import tpu_sc as plsc

NUM_USER = 20000
NUM_ITEM = 30000
N = NUM_USER + NUM_ITEM
N_EDGES = 800000
DIM_FEAT = 128
DIM_LATENT = 64
HALF = DIM_LATENT // 2
N_LAYERS = 3

NC = 2
NS = 16
LANES = 16

NPAD = 51200

CHUNK = 128
CPT = 400
EPT = CPT * CHUNK
E_PAD = NS * EPT
STAGE = 25
GROUP = 5
ACC_ROWS = 50048
RPT = ACC_ROWS // NS
ZBLK = 128

PREP_BLK = 400
PREP_GRID = 2 * NPAD // PREP_BLK
UBLKS = NUM_USER // PREP_BLK
ITEM_BLKS = NUM_ITEM // PREP_BLK


def _prep_body(u_ref, f_ref, w_ref, b_ref, out_ref):
    j = pl.program_id(0)
    i = j % (PREP_GRID // 2)
    t = jnp.dot(f_ref[...], w_ref[...], preferred_element_type=jnp.float32)
    t = t + b_ref[...]
    emb = jnp.where(i < UBLKS, u_ref[...], t)
    nrm = jnp.maximum(jnp.sqrt(jnp.sum(emb * emb, axis=1, keepdims=True)), 1e-12)
    nemb = emb / nrm
    out_ref[...] = jnp.where(j < PREP_GRID // 2, nemb[:, :HALF], nemb[:, HALF:])


def _prep(user_pref, features, W, b2):
    nhalf = PREP_GRID // 2
    return pl.pallas_call(
        _prep_body,
        grid=(PREP_GRID,),
        in_specs=[
            pl.BlockSpec((PREP_BLK, DIM_LATENT),
                         lambda j: (jnp.minimum(j % nhalf, UBLKS - 1), 0)),
            pl.BlockSpec((PREP_BLK, DIM_FEAT),
                         lambda j: (jnp.clip(j % nhalf - UBLKS, 0, ITEM_BLKS - 1), 0)),
            pl.BlockSpec((DIM_FEAT, DIM_LATENT), lambda j: (0, 0)),
            pl.BlockSpec((1, DIM_LATENT), lambda j: (0, 0)),
        ],
        out_specs=pl.BlockSpec((PREP_BLK, HALF), lambda j: (j, 0)),
        out_shape=jax.ShapeDtypeStruct((2 * NPAD, HALF), jnp.float32),
    )(user_pref, features, W, b2)


def _sc_layer_body(tbl_hbm, cols_hbm, rows_hbm, vals_hbm, out_hbm,
                   accum, colst, rowst, valst,
                   rb0, rb1, rb2, rb3, rb4,
                   g0, g1, g2, g3, g4, s0, s1, s2, s3, s4):
    cid = lax.axis_index("c")
    sid = lax.axis_index("s")
    rbufs = (rb0, rb1, rb2, rb3, rb4)
    gsems = (g0, g1, g2, g3, g4)
    ssems = (s0, s1, s2, s3, s4)

    z = jnp.zeros((LANES,), jnp.float32)

    @pl.loop(0, ZBLK)
    def _zero_fill(r):
        rb0[r, pl.ds(0, LANES)] = z
        rb0[r, pl.ds(LANES, LANES)] = z

    @pl.loop(0, RPT // ZBLK)
    def _zero_copy(k):
        pltpu.sync_copy(rb0, accum.at[pl.ds(sid * RPT + k * ZBLK, ZBLK)])

    pltpu.sync_copy(rb0.at[pl.ds(0, RPT % ZBLK)],
                    accum.at[pl.ds(sid * RPT + (RPT // ZBLK) * ZBLK, RPT % ZBLK)])

    plsc.subcore_barrier()

    gps = STAGE // GROUP

    @pl.loop(0, CPT // STAGE)
    def _stage(h):
        pltpu.sync_copy(cols_hbm.at[cid, sid, pl.ds(h * STAGE, STAGE)], colst)
        pltpu.sync_copy(rows_hbm.at[sid, pl.ds(h * STAGE, STAGE)], rowst)
        pltpu.sync_copy(
            vals_hbm.at[sid, pl.ds(h * STAGE * CHUNK, STAGE * CHUNK)], valst)

        for i in range(GROUP - 1):
            pltpu.async_copy(tbl_hbm.at[colst.at[i]], rbufs[i], gsems[i])

        @pl.loop(0, gps)
        def _group(g):
            base = g * GROUP
            for i in range(GROUP):
                j = base + i
                pf = j + GROUP - 1
                ps = (i + GROUP - 1) % GROUP
                if i == 0:
                    @pl.when(g == 0)
                    def _pf0(_pf=pf, _ps=ps):
                        pltpu.async_copy(tbl_hbm.at[colst.at[_pf]],
                                         rbufs[_ps], gsems[_ps])

                    @pl.when(g > 0)
                    def _pfn(_j=j, _pf=pf, _ps=ps):
                        pltpu.make_async_copy(
                            rbufs[_ps], accum.at[rowst.at[_j - 1]],
                            ssems[_ps]).wait()
                        pltpu.async_copy(tbl_hbm.at[colst.at[_pf]],
                                         rbufs[_ps], gsems[_ps])
                else:
                    @pl.when(g < gps - 1)
                    def _pfi(_j=j, _pf=pf, _ps=ps):
                        pltpu.make_async_copy(
                            rbufs[_ps], accum.at[rowst.at[_j - 1]],
                            ssems[_ps]).wait()
                        pltpu.async_copy(tbl_hbm.at[colst.at[_pf]],
                                         rbufs[_ps], gsems[_ps])

                pltpu.make_async_copy(tbl_hbm.at[colst.at[j]], rbufs[i],
                                      gsems[i]).wait()
                jbase = j * CHUNK
                dnums = lax.GatherDimensionNumbers(
                    offset_dims=(), collapsed_slice_dims=(0,),
                    start_index_map=(0,))

                def _scale_body(g16, _i=i, _jbase=jbase, _dnums=dnums):
                    rb = rbufs[_i]
                    vals16 = valst[pl.ds(_jbase + g16 * LANES, LANES)]
                    for l in range(LANES):
                        e = g16 * LANES + l
                        cl = jnp.full((LANES, 1), l, jnp.int32)
                        val = lax.gather(
                            vals16, cl, _dnums, (1,),
                            mode=lax.GatherScatterMode.PROMISE_IN_BOUNDS)
                        rb[e, pl.ds(0, LANES)] = rb[e, pl.ds(0, LANES)] * val
                        rb[e, pl.ds(LANES, LANES)] = (
                            rb[e, pl.ds(LANES, LANES)] * val)

                pl.loop(0, CHUNK // LANES)(_scale_body)
                pltpu.async_copy(rbufs[i], accum.at[rowst.at[j]], ssems[i],
                                 add=True)

        for i in range(GROUP):
            j = STAGE - GROUP + i
            pltpu.make_async_copy(rbufs[i], accum.at[rowst.at[j]],
                                  ssems[i]).wait()

    plsc.subcore_barrier()
    pltpu.sync_copy(
        accum.at[pl.ds(sid * RPT, RPT)],
        out_hbm.at[pl.ds(cid * NPAD + sid * RPT, RPT)])


@functools.lru_cache(maxsize=1)
def _make_sc_layer():
    return pl.kernel(
        _sc_layer_body,
        out_type=jax.ShapeDtypeStruct((2 * NPAD, HALF), jnp.float32),
        mesh=plsc.VectorSubcoreMesh(core_axis_name="c", subcore_axis_name="s",
                                    num_cores=NC, num_subcores=NS),
        scratch_types=[
            pltpu.VMEM_SHARED((ACC_ROWS, HALF), jnp.float32),
            pltpu.VMEM((STAGE, CHUNK), jnp.int32),
            pltpu.VMEM((STAGE, CHUNK), jnp.int32),
            pltpu.VMEM((STAGE * CHUNK,), jnp.float32),
        ] + [pltpu.VMEM((CHUNK, HALF), jnp.float32)] * GROUP
          + [pltpu.SemaphoreType.DMA] * (2 * GROUP),
        compiler_params=pltpu.CompilerParams(needs_layout_passes=False,
                                             use_tc_tiling_on_sc=False),
    )


def _sc_layer(tbl, cols2, rows3, vals3):
    return _make_sc_layer()(tbl, cols2, rows3, vals3)


MCH = 128
MRPT = NPAD // NS


def _sc_merge_body(t0, t1, t2, t3, out_hbm, b0, b1, b2, b3, obuf):
    cid = lax.axis_index("c")
    sid = lax.axis_index("s")

    @pl.loop(0, MRPT // MCH)
    def _chunk(k):
        r0 = cid * NPAD + sid * MRPT + k * MCH
        pltpu.sync_copy(t0.at[pl.ds(r0, MCH)], b0)
        pltpu.sync_copy(t1.at[pl.ds(r0, MCH)], b1)
        pltpu.sync_copy(t2.at[pl.ds(r0, MCH)], b2)
        pltpu.sync_copy(t3.at[pl.ds(r0, MCH)], b3)

        @pl.loop(0, MCH, unroll=4)
        def _row(r):
            for h in (0, LANES):
                s = (b0[r, pl.ds(h, LANES)] + b1[r, pl.ds(h, LANES)]
                     + b2[r, pl.ds(h, LANES)] + b3[r, pl.ds(h, LANES)])
                obuf[r, pl.ds(h, LANES)] = s * 0.25

        pltpu.sync_copy(
            obuf,
            out_hbm.at[pl.ds(sid * MRPT + k * MCH, MCH),
                       pl.ds(cid * HALF, HALF)])


@functools.lru_cache(maxsize=1)
def _make_sc_merge():
    return pl.kernel(
        _sc_merge_body,
        out_type=jax.ShapeDtypeStruct((NPAD, DIM_LATENT), jnp.float32),
        mesh=plsc.VectorSubcoreMesh(core_axis_name="c", subcore_axis_name="s",
                                    num_cores=NC, num_subcores=NS),
        scratch_types=[pltpu.VMEM((MCH, HALF), jnp.float32)] * 5,
        compiler_params=pltpu.CompilerParams(needs_layout_passes=False,
                                             use_tc_tiling_on_sc=False),
    )


def kernel(features, user_id_preference, adj_rows, adj_cols, adj_vals, W, b):
    pad = E_PAD - N_EDGES
    cols_p = jnp.concatenate([adj_cols, jnp.zeros((pad,), jnp.int32)])
    rows_p = jnp.concatenate([adj_rows, jnp.zeros((pad,), jnp.int32)])
    vals_p = jnp.concatenate([adj_vals, jnp.zeros((pad,), jnp.float32)])
    cols2 = jnp.stack([cols_p, cols_p + NPAD]).reshape(NC, NS, CPT, CHUNK)
    rows3 = rows_p.reshape(NS, CPT, CHUNK)
    vals3 = vals_p.reshape(NS, CPT * CHUNK)

    t0 = _prep(user_id_preference, features, W, b.reshape(1, DIM_LATENT))
    t1 = _sc_layer(t0, cols2, rows3, vals3)
    t2 = _sc_layer(t1, cols2, rows3, vals3)
    t3 = _sc_layer(t2, cols2, rows3, vals3)
    out = _make_sc_merge()(t0, t1, t2, t3)
    return (out[:NUM_USER], out[NUM_USER:N])


# speedup vs baseline: 1.0835x; 1.0835x over previous
import functools

import jax
import jax.numpy as jnp
from jax import lax
from jax.experimental import pallas as pl
from jax.experimental.pallas import tpu as pltpu
from jax.experimental.pallas import tpu_sc as plsc

NUM_USER = 20000
NUM_ITEM = 30000
N = NUM_USER + NUM_ITEM
N_EDGES = 800000
DIM_FEAT = 128
DIM_LATENT = 64
HALF = DIM_LATENT // 2
N_LAYERS = 3

NC = 2
NS = 16
LANES = 16

NPAD = 51200

CHUNK = 128
CPT = 400
EPT = CPT * CHUNK
E_PAD = NS * EPT
STAGE = 25
GROUP = 5
ACC_ROWS = 50048
RPT = ACC_ROWS // NS
ZBLK = 128

PREP_BLK = 400
PREP_GRID = 2 * NPAD // PREP_BLK
UBLKS = NUM_USER // PREP_BLK
ITEM_BLKS = NUM_ITEM // PREP_BLK


def _prep_body(u_ref, f_ref, w_ref, b_ref, lo_ref, hi_ref):
    i = pl.program_id(0)
    t = jnp.dot(f_ref[...], w_ref[...], preferred_element_type=jnp.float32)
    t = t + b_ref[...]
    emb = jnp.where(i < UBLKS, u_ref[...], t)
    nrm = jnp.maximum(jnp.sqrt(jnp.sum(emb * emb, axis=1, keepdims=True)), 1e-12)
    nemb = emb / nrm
    lo_ref[...] = nemb[:, :HALF]
    hi_ref[...] = nemb[:, HALF:]


def _prep(user_pref, features, W, b2):
    grid = NPAD // PREP_BLK
    half_spec = pl.BlockSpec((PREP_BLK, HALF), lambda i: (i, 0))
    lo, hi = pl.pallas_call(
        _prep_body,
        grid=(grid,),
        in_specs=[
            pl.BlockSpec((PREP_BLK, DIM_LATENT),
                         lambda i: (jnp.minimum(i, UBLKS - 1), 0)),
            pl.BlockSpec((PREP_BLK, DIM_FEAT),
                         lambda i: (jnp.clip(i - UBLKS, 0, ITEM_BLKS - 1), 0)),
            pl.BlockSpec((DIM_FEAT, DIM_LATENT), lambda i: (0, 0)),
            pl.BlockSpec((1, DIM_LATENT), lambda i: (0, 0)),
        ],
        out_specs=[half_spec, half_spec],
        out_shape=[jax.ShapeDtypeStruct((NPAD, HALF), jnp.float32)] * 2,
    )(user_pref, features, W, b2)
    return jnp.concatenate([lo, hi], axis=0)


def _sc_layer_body(tbl_hbm, cols_hbm, rows_hbm, vals_hbm, out_hbm,
                   accum, colst, rowst, valst,
                   rb0, rb1, rb2, rb3, rb4,
                   g0, g1, g2, g3, g4, s0, s1, s2, s3, s4):
    cid = lax.axis_index("c")
    sid = lax.axis_index("s")
    rbufs = (rb0, rb1, rb2, rb3, rb4)
    gsems = (g0, g1, g2, g3, g4)
    ssems = (s0, s1, s2, s3, s4)

    z = jnp.zeros((LANES,), jnp.float32)

    @pl.loop(0, ZBLK)
    def _zero_fill(r):
        rb0[r, pl.ds(0, LANES)] = z
        rb0[r, pl.ds(LANES, LANES)] = z

    @pl.loop(0, RPT // ZBLK)
    def _zero_copy(k):
        pltpu.sync_copy(rb0, accum.at[pl.ds(sid * RPT + k * ZBLK, ZBLK)])

    pltpu.sync_copy(rb0.at[pl.ds(0, RPT % ZBLK)],
                    accum.at[pl.ds(sid * RPT + (RPT // ZBLK) * ZBLK, RPT % ZBLK)])

    plsc.subcore_barrier()

    gps = STAGE // GROUP

    @pl.loop(0, CPT // STAGE)
    def _stage(h):
        pltpu.sync_copy(cols_hbm.at[cid, sid, pl.ds(h * STAGE, STAGE)], colst)
        pltpu.sync_copy(rows_hbm.at[sid, pl.ds(h * STAGE, STAGE)], rowst)
        pltpu.sync_copy(
            vals_hbm.at[sid, pl.ds(h * STAGE * CHUNK, STAGE * CHUNK)], valst)

        for i in range(GROUP - 1):
            pltpu.async_copy(tbl_hbm.at[colst.at[i]], rbufs[i], gsems[i])

        @pl.loop(0, gps)
        def _group(g):
            base = g * GROUP
            for i in range(GROUP):
                j = base + i
                pf = j + GROUP - 1
                ps = (i + GROUP - 1) % GROUP
                if i == 0:
                    @pl.when(g == 0)
                    def _pf0(_pf=pf, _ps=ps):
                        pltpu.async_copy(tbl_hbm.at[colst.at[_pf]],
                                         rbufs[_ps], gsems[_ps])

                    @pl.when(g > 0)
                    def _pfn(_j=j, _pf=pf, _ps=ps):
                        pltpu.make_async_copy(
                            rbufs[_ps], accum.at[rowst.at[_j - 1]],
                            ssems[_ps]).wait()
                        pltpu.async_copy(tbl_hbm.at[colst.at[_pf]],
                                         rbufs[_ps], gsems[_ps])
                else:
                    @pl.when(g < gps - 1)
                    def _pfi(_j=j, _pf=pf, _ps=ps):
                        pltpu.make_async_copy(
                            rbufs[_ps], accum.at[rowst.at[_j - 1]],
                            ssems[_ps]).wait()
                        pltpu.async_copy(tbl_hbm.at[colst.at[_pf]],
                                         rbufs[_ps], gsems[_ps])

                pltpu.make_async_copy(tbl_hbm.at[colst.at[j]], rbufs[i],
                                      gsems[i]).wait()
                jbase = j * CHUNK
                dnums = lax.GatherDimensionNumbers(
                    offset_dims=(), collapsed_slice_dims=(0,),
                    start_index_map=(0,))

                def _scale_body(g16, _i=i, _jbase=jbase, _dnums=dnums):
                    rb = rbufs[_i]
                    vals16 = valst[pl.ds(_jbase + g16 * LANES, LANES)]
                    for l in range(LANES):
                        e = g16 * LANES + l
                        cl = jnp.full((LANES, 1), l, jnp.int32)
                        val = lax.gather(
                            vals16, cl, _dnums, (1,),
                            mode=lax.GatherScatterMode.PROMISE_IN_BOUNDS)
                        rb[e, pl.ds(0, LANES)] = rb[e, pl.ds(0, LANES)] * val
                        rb[e, pl.ds(LANES, LANES)] = (
                            rb[e, pl.ds(LANES, LANES)] * val)

                pl.loop(0, CHUNK // LANES)(_scale_body)
                pltpu.async_copy(rbufs[i], accum.at[rowst.at[j]], ssems[i],
                                 add=True)

        for i in range(GROUP):
            j = STAGE - GROUP + i
            pltpu.make_async_copy(rbufs[i], accum.at[rowst.at[j]],
                                  ssems[i]).wait()

    plsc.subcore_barrier()
    pltpu.sync_copy(
        accum.at[pl.ds(sid * RPT, RPT)],
        out_hbm.at[pl.ds(cid * NPAD + sid * RPT, RPT)])


@functools.lru_cache(maxsize=1)
def _make_sc_layer():
    return pl.kernel(
        _sc_layer_body,
        out_type=jax.ShapeDtypeStruct((2 * NPAD, HALF), jnp.float32),
        mesh=plsc.VectorSubcoreMesh(core_axis_name="c", subcore_axis_name="s",
                                    num_cores=NC, num_subcores=NS),
        scratch_types=[
            pltpu.VMEM_SHARED((ACC_ROWS, HALF), jnp.float32),
            pltpu.VMEM((STAGE, CHUNK), jnp.int32),
            pltpu.VMEM((STAGE, CHUNK), jnp.int32),
            pltpu.VMEM((STAGE * CHUNK,), jnp.float32),
        ] + [pltpu.VMEM((CHUNK, HALF), jnp.float32)] * GROUP
          + [pltpu.SemaphoreType.DMA] * (2 * GROUP),
        compiler_params=pltpu.CompilerParams(needs_layout_passes=False,
                                             use_tc_tiling_on_sc=False),
    )


def _sc_layer(tbl, cols2, rows3, vals3):
    return _make_sc_layer()(tbl, cols2, rows3, vals3)


MCH = 128
MRPT = NPAD // NS


def _sc_merge_body(t0, t1, t2, t3, out_hbm, b0, b1, b2, b3, obuf):
    cid = lax.axis_index("c")
    sid = lax.axis_index("s")

    @pl.loop(0, MRPT // MCH)
    def _chunk(k):
        r0 = cid * NPAD + sid * MRPT + k * MCH
        pltpu.sync_copy(t0.at[pl.ds(r0, MCH)], b0)
        pltpu.sync_copy(t1.at[pl.ds(r0, MCH)], b1)
        pltpu.sync_copy(t2.at[pl.ds(r0, MCH)], b2)
        pltpu.sync_copy(t3.at[pl.ds(r0, MCH)], b3)

        @pl.loop(0, MCH, unroll=4)
        def _row(r):
            for h in (0, LANES):
                s = (b0[r, pl.ds(h, LANES)] + b1[r, pl.ds(h, LANES)]
                     + b2[r, pl.ds(h, LANES)] + b3[r, pl.ds(h, LANES)])
                obuf[r, pl.ds(h, LANES)] = s * 0.25

        pltpu.sync_copy(
            obuf,
            out_hbm.at[pl.ds(sid * MRPT + k * MCH, MCH),
                       pl.ds(cid * HALF, HALF)])


@functools.lru_cache(maxsize=1)
def _make_sc_merge():
    return pl.kernel(
        _sc_merge_body,
        out_type=jax.ShapeDtypeStruct((NPAD, DIM_LATENT), jnp.float32),
        mesh=plsc.VectorSubcoreMesh(core_axis_name="c", subcore_axis_name="s",
                                    num_cores=NC, num_subcores=NS),
        scratch_types=[pltpu.VMEM((MCH, HALF), jnp.float32)] * 5,
        compiler_params=pltpu.CompilerParams(needs_layout_passes=False,
                                             use_tc_tiling_on_sc=False),
    )


def kernel(features, user_id_preference, adj_rows, adj_cols, adj_vals, W, b):
    pad = E_PAD - N_EDGES
    cols_p = jnp.concatenate([adj_cols, jnp.zeros((pad,), jnp.int32)])
    rows_p = jnp.concatenate([adj_rows, jnp.zeros((pad,), jnp.int32)])
    vals_p = jnp.concatenate([adj_vals, jnp.zeros((pad,), jnp.float32)])
    cols2 = jnp.stack([cols_p, cols_p + NPAD]).reshape(NC, NS, CPT, CHUNK)
    rows3 = rows_p.reshape(NS, CPT, CHUNK)
    vals3 = vals_p.reshape(NS, CPT * CHUNK)

    t0 = _prep(user_id_preference, features, W, b.reshape(1, DIM_LATENT))
    t1 = _sc_layer(t0, cols2, rows3, vals3)
    t2 = _sc_layer(t1, cols2, rows3, vals3)
    t3 = _sc_layer(t2, cols2, rows3, vals3)
    out = _make_sc_merge()(t0, t1, t2, t3)
    return (out[:NUM_USER], out[NUM_USER:N])


# speedup vs baseline: 1.1251x; 1.0384x over previous
import functools

import jax
import jax.numpy as jnp
from jax import lax
from jax.experimental import pallas as pl
from jax.experimental.pallas import tpu as pltpu
from jax.experimental.pallas import tpu_sc as plsc

NUM_USER = 20000
NUM_ITEM = 30000
N = NUM_USER + NUM_ITEM
N_EDGES = 800000
DIM_FEAT = 128
DIM_LATENT = 64
HALF = DIM_LATENT // 2
N_LAYERS = 3

NC = 2
NS = 16
LANES = 16

NPAD = 51200

CHUNK = 128
CPT = 400
EPT = CPT * CHUNK
E_PAD = NS * EPT
STAGE = 25
GROUP = 5
ACC_ROWS = 50048
RPT = ACC_ROWS // NS
ZBLK = 128

PREP_BLK = 400
PREP_GRID = 2 * NPAD // PREP_BLK
UBLKS = NUM_USER // PREP_BLK
ITEM_BLKS = NUM_ITEM // PREP_BLK


def _prep_body(u_ref, f_ref, w_ref, b_ref, lo_ref, hi_ref):
    i = pl.program_id(0)
    t = jnp.dot(f_ref[...], w_ref[...], preferred_element_type=jnp.float32)
    t = t + b_ref[...]
    emb = jnp.where(i < UBLKS, u_ref[...], t)
    nrm = jnp.maximum(jnp.sqrt(jnp.sum(emb * emb, axis=1, keepdims=True)), 1e-12)
    nemb = emb / nrm
    lo_ref[...] = nemb[:, :HALF]
    hi_ref[...] = nemb[:, HALF:]


def _prep(user_pref, features, W, b2):
    grid = NPAD // PREP_BLK
    half_spec = pl.BlockSpec((PREP_BLK, HALF), lambda i: (i, 0))
    lo, hi = pl.pallas_call(
        _prep_body,
        grid=(grid,),
        in_specs=[
            pl.BlockSpec((PREP_BLK, DIM_LATENT),
                         lambda i: (jnp.minimum(i, UBLKS - 1), 0)),
            pl.BlockSpec((PREP_BLK, DIM_FEAT),
                         lambda i: (jnp.clip(i - UBLKS, 0, ITEM_BLKS - 1), 0)),
            pl.BlockSpec((DIM_FEAT, DIM_LATENT), lambda i: (0, 0)),
            pl.BlockSpec((1, DIM_LATENT), lambda i: (0, 0)),
        ],
        out_specs=[half_spec, half_spec],
        out_shape=[jax.ShapeDtypeStruct((NPAD, HALF), jnp.float32)] * 2,
    )(user_pref, features, W, b2)
    return jnp.concatenate([lo, hi], axis=0)


def _sc_layer_body(tbl_hbm, cols_hbm, rows_hbm, vals_hbm, out_hbm,
                   accum, colst, rowst, valst,
                   rb0, rb1, rb2, rb3, rb4,
                   g0, g1, g2, g3, g4, s0, s1, s2, s3, s4):
    cid = lax.axis_index("c")
    sid = lax.axis_index("s")
    rbufs = (rb0, rb1, rb2, rb3, rb4)
    gsems = (g0, g1, g2, g3, g4)
    ssems = (s0, s1, s2, s3, s4)

    z = jnp.zeros((LANES,), jnp.float32)

    @pl.loop(0, ZBLK)
    def _zero_fill(r):
        rb0[r, pl.ds(0, LANES)] = z
        rb0[r, pl.ds(LANES, LANES)] = z

    @pl.loop(0, RPT // ZBLK)
    def _zero_copy(k):
        pltpu.sync_copy(rb0, accum.at[pl.ds(sid * RPT + k * ZBLK, ZBLK)])

    pltpu.sync_copy(rb0.at[pl.ds(0, RPT % ZBLK)],
                    accum.at[pl.ds(sid * RPT + (RPT // ZBLK) * ZBLK, RPT % ZBLK)])

    plsc.subcore_barrier()

    gps = STAGE // GROUP

    @pl.loop(0, CPT // STAGE)
    def _stage(h):
        pltpu.sync_copy(cols_hbm.at[cid, sid, pl.ds(h * STAGE, STAGE)], colst)
        pltpu.sync_copy(rows_hbm.at[sid, pl.ds(h * STAGE, STAGE)], rowst)
        pltpu.sync_copy(
            vals_hbm.at[sid, pl.ds(h * STAGE * CHUNK, STAGE * CHUNK)], valst)

        for i in range(GROUP - 1):
            pltpu.async_copy(tbl_hbm.at[colst.at[i]], rbufs[i], gsems[i])

        @pl.loop(0, gps)
        def _group(g):
            base = g * GROUP
            for i in range(GROUP):
                j = base + i
                pf = j + GROUP - 1
                ps = (i + GROUP - 1) % GROUP
                if i == 0:
                    @pl.when(g == 0)
                    def _pf0(_pf=pf, _ps=ps):
                        pltpu.async_copy(tbl_hbm.at[colst.at[_pf]],
                                         rbufs[_ps], gsems[_ps])

                    @pl.when(g > 0)
                    def _pfn(_j=j, _pf=pf, _ps=ps):
                        pltpu.make_async_copy(
                            rbufs[_ps], accum.at[rowst.at[_j - 1]],
                            ssems[_ps]).wait()
                        pltpu.async_copy(tbl_hbm.at[colst.at[_pf]],
                                         rbufs[_ps], gsems[_ps])
                else:
                    @pl.when(g < gps - 1)
                    def _pfi(_j=j, _pf=pf, _ps=ps):
                        pltpu.make_async_copy(
                            rbufs[_ps], accum.at[rowst.at[_j - 1]],
                            ssems[_ps]).wait()
                        pltpu.async_copy(tbl_hbm.at[colst.at[_pf]],
                                         rbufs[_ps], gsems[_ps])

                pltpu.make_async_copy(tbl_hbm.at[colst.at[j]], rbufs[i],
                                      gsems[i]).wait()
                jbase = j * CHUNK
                dnums = lax.GatherDimensionNumbers(
                    offset_dims=(), collapsed_slice_dims=(0,),
                    start_index_map=(0,))

                def _scale_body(g16, _i=i, _jbase=jbase, _dnums=dnums):
                    rb = rbufs[_i]
                    vals16 = valst[pl.ds(_jbase + g16 * LANES, LANES)]
                    for l in range(LANES):
                        e = g16 * LANES + l
                        cl = jnp.full((LANES, 1), l, jnp.int32)
                        val = lax.gather(
                            vals16, cl, _dnums, (1,),
                            mode=lax.GatherScatterMode.PROMISE_IN_BOUNDS)
                        rb[e, pl.ds(0, LANES)] = rb[e, pl.ds(0, LANES)] * val
                        rb[e, pl.ds(LANES, LANES)] = (
                            rb[e, pl.ds(LANES, LANES)] * val)

                pl.loop(0, CHUNK // LANES)(_scale_body)
                pltpu.async_copy(rbufs[i], accum.at[rowst.at[j]], ssems[i],
                                 add=True)

        for i in range(GROUP):
            j = STAGE - GROUP + i
            pltpu.make_async_copy(rbufs[i], accum.at[rowst.at[j]],
                                  ssems[i]).wait()

    plsc.subcore_barrier()
    pltpu.sync_copy(
        accum.at[pl.ds(sid * RPT, RPT)],
        out_hbm.at[pl.ds(cid * NPAD + sid * RPT, RPT)])


@functools.lru_cache(maxsize=1)
def _make_sc_layer():
    return pl.kernel(
        _sc_layer_body,
        out_type=jax.ShapeDtypeStruct((2 * NPAD, HALF), jnp.float32),
        mesh=plsc.VectorSubcoreMesh(core_axis_name="c", subcore_axis_name="s",
                                    num_cores=NC, num_subcores=NS),
        scratch_types=[
            pltpu.VMEM_SHARED((ACC_ROWS, HALF), jnp.float32),
            pltpu.VMEM((STAGE, CHUNK), jnp.int32),
            pltpu.VMEM((STAGE, CHUNK), jnp.int32),
            pltpu.VMEM((STAGE * CHUNK,), jnp.float32),
        ] + [pltpu.VMEM((CHUNK, HALF), jnp.float32)] * GROUP
          + [pltpu.SemaphoreType.DMA] * (2 * GROUP),
        compiler_params=pltpu.CompilerParams(needs_layout_passes=False,
                                             use_tc_tiling_on_sc=False),
    )


def _sc_layer(tbl, cols2, rows3, vals3):
    return _make_sc_layer()(tbl, cols2, rows3, vals3)


MCH = 128
MRPT = NPAD // NS


def _sc_merge_body(t0, t1, t2, t3, out_hbm,
                   a0, a1, a2, a3, b0, b1, b2, b3, obuf, sa, sb):
    cid = lax.axis_index("c")
    sid = lax.axis_index("s")
    tabs = (t0, t1, t2, t3)
    sets = ((a0, a1, a2, a3, sa), (b0, b1, b2, b3, sb))
    nch = MRPT // MCH

    def fire(k):
        bufs = sets[k % 2]
        r0 = cid * NPAD + sid * MRPT + k * MCH
        for t, bf in zip(tabs, bufs[:4]):
            pltpu.async_copy(t.at[pl.ds(r0, MCH)], bf, bufs[4])

    def drain_compute_store(k):
        bufs = sets[k % 2]
        r0 = cid * NPAD + sid * MRPT + k * MCH
        for t, bf in zip(tabs, bufs[:4]):
            pltpu.make_async_copy(t.at[pl.ds(r0, MCH)], bf, bufs[4]).wait()
        c0, c1, c2, c3 = bufs[:4]

        def _row(r):
            for h in (0, LANES):
                s = (c0[r, pl.ds(h, LANES)] + c1[r, pl.ds(h, LANES)]
                     + c2[r, pl.ds(h, LANES)] + c3[r, pl.ds(h, LANES)])
                obuf[r, pl.ds(h, LANES)] = s * 0.25

        pl.loop(0, MCH, unroll=4)(_row)
        pltpu.sync_copy(
            obuf,
            out_hbm.at[pl.ds(sid * MRPT + k * MCH, MCH),
                       pl.ds(cid * HALF, HALF)])

    fire(0)
    for k in range(nch):
        if k + 1 < nch:
            fire(k + 1)
        drain_compute_store(k)


@functools.lru_cache(maxsize=1)
def _make_sc_merge():
    return pl.kernel(
        _sc_merge_body,
        out_type=jax.ShapeDtypeStruct((NPAD, DIM_LATENT), jnp.float32),
        mesh=plsc.VectorSubcoreMesh(core_axis_name="c", subcore_axis_name="s",
                                    num_cores=NC, num_subcores=NS),
        scratch_types=[pltpu.VMEM((MCH, HALF), jnp.float32)] * 9
          + [pltpu.SemaphoreType.DMA] * 2,
        compiler_params=pltpu.CompilerParams(needs_layout_passes=False,
                                             use_tc_tiling_on_sc=False),
    )


def kernel(features, user_id_preference, adj_rows, adj_cols, adj_vals, W, b):
    pad = E_PAD - N_EDGES
    cols_p = jnp.concatenate([adj_cols, jnp.zeros((pad,), jnp.int32)])
    rows_p = jnp.concatenate([adj_rows, jnp.zeros((pad,), jnp.int32)])
    vals_p = jnp.concatenate([adj_vals, jnp.zeros((pad,), jnp.float32)])
    cols2 = jnp.stack([cols_p, cols_p + NPAD]).reshape(NC, NS, CPT, CHUNK)
    rows3 = rows_p.reshape(NS, CPT, CHUNK)
    vals3 = vals_p.reshape(NS, CPT * CHUNK)

    t0 = _prep(user_id_preference, features, W, b.reshape(1, DIM_LATENT))
    t1 = _sc_layer(t0, cols2, rows3, vals3)
    t2 = _sc_layer(t1, cols2, rows3, vals3)
    t3 = _sc_layer(t2, cols2, rows3, vals3)
    out = _make_sc_merge()(t0, t1, t2, t3)
    return (out[:NUM_USER], out[NUM_USER:N])


# speedup vs baseline: 1.1734x; 1.0429x over previous
import functools

import jax
import jax.numpy as jnp
from jax import lax
from jax.experimental import pallas as pl
from jax.experimental.pallas import tpu as pltpu
from jax.experimental.pallas import tpu_sc as plsc

NUM_USER = 20000
NUM_ITEM = 30000
N = NUM_USER + NUM_ITEM
N_EDGES = 800000
DIM_FEAT = 128
DIM_LATENT = 64
HALF = DIM_LATENT // 2
N_LAYERS = 3

NC = 2
NS = 16
LANES = 16

NPAD = 51200

CHUNK = 128
CPT = 400
EPT = CPT * CHUNK
E_PAD = NS * EPT
STAGE = 25
GROUP = 5
ACC_ROWS = 50048
RPT = ACC_ROWS // NS
ZBLK = 128

PREP_BLK = 400
PREP_GRID = 2 * NPAD // PREP_BLK
UBLKS = NUM_USER // PREP_BLK
ITEM_BLKS = NUM_ITEM // PREP_BLK


def _prep_body(u_ref, f_ref, w_ref, b_ref, lo_ref, hi_ref):
    i = pl.program_id(0)
    t = jnp.dot(f_ref[...], w_ref[...], preferred_element_type=jnp.float32)
    t = t + b_ref[...]
    emb = jnp.where(i < UBLKS, u_ref[...], t)
    nrm = jnp.maximum(jnp.sqrt(jnp.sum(emb * emb, axis=1, keepdims=True)), 1e-12)
    nemb = emb / nrm
    lo_ref[...] = nemb[:, :HALF]
    hi_ref[...] = nemb[:, HALF:]


def _prep(user_pref, features, W, b2):
    grid = NPAD // PREP_BLK
    half_spec = pl.BlockSpec((PREP_BLK, HALF), lambda i: (i, 0))
    lo, hi = pl.pallas_call(
        _prep_body,
        grid=(grid,),
        in_specs=[
            pl.BlockSpec((PREP_BLK, DIM_LATENT),
                         lambda i: (jnp.minimum(i, UBLKS - 1), 0)),
            pl.BlockSpec((PREP_BLK, DIM_FEAT),
                         lambda i: (jnp.clip(i - UBLKS, 0, ITEM_BLKS - 1), 0)),
            pl.BlockSpec((DIM_FEAT, DIM_LATENT), lambda i: (0, 0)),
            pl.BlockSpec((1, DIM_LATENT), lambda i: (0, 0)),
        ],
        out_specs=[half_spec, half_spec],
        out_shape=[jax.ShapeDtypeStruct((NPAD, HALF), jnp.float32)] * 2,
    )(user_pref, features, W, b2)
    return jnp.concatenate([lo, hi], axis=0)


def _sc_layer_body(tbl_hbm, cols_hbm, rows_hbm, vals_hbm, out_hbm,
                   accum, colst, rowst, valst,
                   rb0, rb1, rb2, rb3, rb4,
                   g0, g1, g2, g3, g4, s0, s1, s2, s3, s4):
    cid = lax.axis_index("c")
    sid = lax.axis_index("s")
    rbufs = (rb0, rb1, rb2, rb3, rb4)
    gsems = (g0, g1, g2, g3, g4)
    ssems = (s0, s1, s2, s3, s4)

    z = jnp.zeros((LANES,), jnp.float32)

    @pl.loop(0, ZBLK)
    def _zero_fill(r):
        rb0[r, pl.ds(0, LANES)] = z
        rb0[r, pl.ds(LANES, LANES)] = z

    @pl.loop(0, RPT // ZBLK)
    def _zero_copy(k):
        pltpu.sync_copy(rb0, accum.at[pl.ds(sid * RPT + k * ZBLK, ZBLK)])

    pltpu.sync_copy(rb0.at[pl.ds(0, RPT % ZBLK)],
                    accum.at[pl.ds(sid * RPT + (RPT // ZBLK) * ZBLK, RPT % ZBLK)])

    plsc.subcore_barrier()

    gps = STAGE // GROUP

    @pl.loop(0, CPT // STAGE)
    def _stage(h):
        pltpu.sync_copy(
            cols_hbm.at[pl.ds((cid * NS + sid) * EPT + h * STAGE * CHUNK,
                              STAGE * CHUNK)], colst)
        pltpu.sync_copy(rows_hbm.at[sid, pl.ds(h * STAGE, STAGE)], rowst)
        pltpu.sync_copy(
            vals_hbm.at[pl.ds(sid * EPT + h * STAGE * CHUNK, STAGE * CHUNK)],
            valst)

        for i in range(GROUP - 1):
            pltpu.async_copy(tbl_hbm.at[colst.at[pl.ds(i * CHUNK, CHUNK)]],
                             rbufs[i], gsems[i])

        @pl.loop(0, gps)
        def _group(g):
            base = g * GROUP
            for i in range(GROUP):
                j = base + i
                pf = j + GROUP - 1
                ps = (i + GROUP - 1) % GROUP
                if i == 0:
                    @pl.when(g == 0)
                    def _pf0(_pf=pf, _ps=ps):
                        pltpu.async_copy(
                            tbl_hbm.at[colst.at[pl.ds(_pf * CHUNK, CHUNK)]],
                            rbufs[_ps], gsems[_ps])

                    @pl.when(g > 0)
                    def _pfn(_j=j, _pf=pf, _ps=ps):
                        pltpu.make_async_copy(
                            rbufs[_ps], accum.at[rowst.at[_j - 1]],
                            ssems[_ps]).wait()
                        pltpu.async_copy(
                            tbl_hbm.at[colst.at[pl.ds(_pf * CHUNK, CHUNK)]],
                            rbufs[_ps], gsems[_ps])
                else:
                    @pl.when(g < gps - 1)
                    def _pfi(_j=j, _pf=pf, _ps=ps):
                        pltpu.make_async_copy(
                            rbufs[_ps], accum.at[rowst.at[_j - 1]],
                            ssems[_ps]).wait()
                        pltpu.async_copy(
                            tbl_hbm.at[colst.at[pl.ds(_pf * CHUNK, CHUNK)]],
                            rbufs[_ps], gsems[_ps])

                pltpu.make_async_copy(
                    tbl_hbm.at[colst.at[pl.ds(j * CHUNK, CHUNK)]], rbufs[i],
                    gsems[i]).wait()
                jbase = j * CHUNK
                dnums = lax.GatherDimensionNumbers(
                    offset_dims=(), collapsed_slice_dims=(0,),
                    start_index_map=(0,))

                def _scale_body(g16, _i=i, _jbase=jbase, _dnums=dnums):
                    rb = rbufs[_i]
                    vals16 = valst[pl.ds(_jbase + g16 * LANES, LANES)]
                    for l in range(LANES):
                        e = g16 * LANES + l
                        cl = jnp.full((LANES, 1), l, jnp.int32)
                        val = lax.gather(
                            vals16, cl, _dnums, (1,),
                            mode=lax.GatherScatterMode.PROMISE_IN_BOUNDS)
                        rb[e, pl.ds(0, LANES)] = rb[e, pl.ds(0, LANES)] * val
                        rb[e, pl.ds(LANES, LANES)] = (
                            rb[e, pl.ds(LANES, LANES)] * val)

                pl.loop(0, CHUNK // LANES)(_scale_body)
                pltpu.async_copy(rbufs[i], accum.at[rowst.at[j]], ssems[i],
                                 add=True)

        for i in range(GROUP):
            j = STAGE - GROUP + i
            pltpu.make_async_copy(rbufs[i], accum.at[rowst.at[j]],
                                  ssems[i]).wait()

    plsc.subcore_barrier()
    pltpu.sync_copy(
        accum.at[pl.ds(sid * RPT, RPT)],
        out_hbm.at[pl.ds(cid * NPAD + sid * RPT, RPT)])


@functools.lru_cache(maxsize=1)
def _make_sc_layer():
    return pl.kernel(
        _sc_layer_body,
        out_type=jax.ShapeDtypeStruct((2 * NPAD, HALF), jnp.float32),
        mesh=plsc.VectorSubcoreMesh(core_axis_name="c", subcore_axis_name="s",
                                    num_cores=NC, num_subcores=NS),
        scratch_types=[
            pltpu.VMEM_SHARED((ACC_ROWS, HALF), jnp.float32),
            pltpu.VMEM((STAGE * CHUNK,), jnp.int32),
            pltpu.VMEM((STAGE, CHUNK), jnp.int32),
            pltpu.VMEM((STAGE * CHUNK,), jnp.float32),
        ] + [pltpu.VMEM((CHUNK, HALF), jnp.float32)] * GROUP
          + [pltpu.SemaphoreType.DMA] * (2 * GROUP),
        compiler_params=pltpu.CompilerParams(needs_layout_passes=False,
                                             use_tc_tiling_on_sc=False),
    )


def _sc_layer(tbl, cols2, rows3, vals3):
    return _make_sc_layer()(tbl, cols2, rows3, vals3)


MCH = 128
MRPT = NPAD // NS


def _sc_merge_body(t0, t1, t2, t3, out_hbm,
                   a0, a1, a2, a3, b0, b1, b2, b3, obuf, sa, sb):
    cid = lax.axis_index("c")
    sid = lax.axis_index("s")
    tabs = (t0, t1, t2, t3)
    sets = ((a0, a1, a2, a3, sa), (b0, b1, b2, b3, sb))
    nch = MRPT // MCH

    def fire(k):
        bufs = sets[k % 2]
        r0 = cid * NPAD + sid * MRPT + k * MCH
        for t, bf in zip(tabs, bufs[:4]):
            pltpu.async_copy(t.at[pl.ds(r0, MCH)], bf, bufs[4])

    def drain_compute_store(k):
        bufs = sets[k % 2]
        r0 = cid * NPAD + sid * MRPT + k * MCH
        for t, bf in zip(tabs, bufs[:4]):
            pltpu.make_async_copy(t.at[pl.ds(r0, MCH)], bf, bufs[4]).wait()
        c0, c1, c2, c3 = bufs[:4]

        def _row(r):
            for h in (0, LANES):
                s = (c0[r, pl.ds(h, LANES)] + c1[r, pl.ds(h, LANES)]
                     + c2[r, pl.ds(h, LANES)] + c3[r, pl.ds(h, LANES)])
                obuf[r, pl.ds(h, LANES)] = s * 0.25

        pl.loop(0, MCH, unroll=4)(_row)
        pltpu.sync_copy(
            obuf,
            out_hbm.at[pl.ds(sid * MRPT + k * MCH, MCH),
                       pl.ds(cid * HALF, HALF)])

    fire(0)
    for k in range(nch):
        if k + 1 < nch:
            fire(k + 1)
        drain_compute_store(k)


@functools.lru_cache(maxsize=1)
def _make_sc_merge():
    return pl.kernel(
        _sc_merge_body,
        out_type=jax.ShapeDtypeStruct((NPAD, DIM_LATENT), jnp.float32),
        mesh=plsc.VectorSubcoreMesh(core_axis_name="c", subcore_axis_name="s",
                                    num_cores=NC, num_subcores=NS),
        scratch_types=[pltpu.VMEM((MCH, HALF), jnp.float32)] * 9
          + [pltpu.SemaphoreType.DMA] * 2,
        compiler_params=pltpu.CompilerParams(needs_layout_passes=False,
                                             use_tc_tiling_on_sc=False),
    )


def kernel(features, user_id_preference, adj_rows, adj_cols, adj_vals, W, b):
    pad = E_PAD - N_EDGES
    cols_p = jnp.concatenate([adj_cols, jnp.zeros((pad,), jnp.int32)])
    rows_p = jnp.concatenate([adj_rows, jnp.zeros((pad,), jnp.int32)])
    vals_p = jnp.concatenate([adj_vals, jnp.zeros((pad,), jnp.float32)])
    cols2 = jnp.stack([cols_p, cols_p + NPAD]).reshape(-1)
    rows3 = rows_p.reshape(NS, CPT, CHUNK)
    vals3 = vals_p

    t0 = _prep(user_id_preference, features, W, b.reshape(1, DIM_LATENT))
    t1 = _sc_layer(t0, cols2, rows3, vals3)
    t2 = _sc_layer(t1, cols2, rows3, vals3)
    t3 = _sc_layer(t2, cols2, rows3, vals3)
    out = _make_sc_merge()(t0, t1, t2, t3)
    return (out[:NUM_USER], out[NUM_USER:N])
